# baseline (speedup 1.0000x reference)
import functools

import jax
import jax.numpy as jnp
from jax import lax
from jax.experimental import pallas as pl
from jax.experimental.pallas import tpu as pltpu
from jax.experimental.pallas import tpu_sc as plsc

G = 1024
CHUNK = 32
NBUF = 4
NHP = 2
NW = 32


def _lane_bcast(v16, i):
    return lax.gather(
        v16,
        jnp.full((16, 1), i, jnp.int32),
        lax.GatherDimensionNumbers(
            offset_dims=(),
            collapsed_slice_dims=(0,),
            start_index_map=(0,),
        ),
        slice_sizes=(1,),
        mode=lax.GatherScatterMode.PROMISE_IN_BOUNDS,
    )


def _sc_body(B, S, H, feats_hbm, gids_hbm, vals_hbm, out_hbm,
             bufs, gscan, vscan, acc, sems):
    c = lax.axis_index("c")
    s = lax.axis_index("s")
    w = c * 16 + s

    wpb = NW // B
    gpw = G // wpb
    HP = H // NHP
    hv = HP // 16
    nch = S // CHUNK

    bw = w // wpb
    gr0 = (w % wpb) * gpw
    tok_base = bw * S

    pltpu.sync_copy(gids_hbm.at[pl.ds(tok_base, S)], gscan)
    pltpu.sync_copy(vals_hbm.at[pl.ds(tok_base, S)], vscan)
    lo_t = jnp.full((16,), gr0, jnp.int32)
    hi_t = jnp.full((16,), gr0 + gpw, jnp.int32)
    one = jnp.ones((16,), jnp.int32)
    zero = jnp.zeros((16,), jnp.int32)

    def count_step(i, carry):
        lo, hi = carry
        for u in range(4):
            v = gscan[pl.ds(i * 64 + u * 16, 16)]
            lo = lo + jnp.where(v < lo_t, one, zero)
            hi = hi + jnp.where(v < hi_t, one, zero)
        return lo, hi

    lo_cnt, hi_cnt = lax.fori_loop(0, S // 64, count_step, (zero, zero))
    s_lo = jnp.int32(0)
    s_hi = jnp.int32(0)
    for j in range(16):
        s_lo = s_lo + lo_cnt[j]
        s_hi = s_hi + hi_cnt[j]

    p_lo = s_lo // (NBUF * CHUNK)
    p_hi = (s_hi + NBUF * CHUNK - 1) // (NBUF * CHUNK)
    last_ch = nch - 1
    zvec = jnp.zeros((16,), jnp.float32)
    lo16 = jnp.full((16,), s_lo, jnp.int32)
    hi16 = jnp.full((16,), s_hi, jnp.int32)
    gr0_16 = jnp.full((16,), gr0, jnp.int32)

    for hp in range(NHP):
        col0 = hp * HP

        @plsc.parallel_loop(0, gpw, 1, unroll=4)
        def zero_row(t):
            for k in range(hv):
                acc[t, pl.ds(k * 16, 16)] = zvec

        def issue(ch, buf, sem):
            chc = jnp.minimum(ch, last_ch)
            pltpu.async_copy(
                feats_hbm.at[pl.ds(tok_base + chc * CHUNK, CHUNK),
                             pl.ds(col0, HP)],
                buf, sem)

        def wait(buf, sem):
            pltpu.make_async_copy(
                feats_hbm.at[pl.ds(0, CHUNK), pl.ds(col0, HP)],
                buf, sem).wait()

        lane = lax.iota(jnp.int32, 16)

        def compute(ch, buf):
            def block_step(jj, _):
                off = ch * CHUNK + jj * 16
                g16 = gscan[pl.ds(off, 16)]
                v16 = vscan[pl.ds(off, 16)]
                tok16 = off + lane
                m = (tok16 >= lo16) & (tok16 < hi16)
                val16 = jnp.where(m, v16, 0.0)
                r16 = jnp.clip(g16 - gr0_16, 0, gpw - 1)
                valvs = [_lane_bcast(val16, i) for i in range(16)]
                rvecs = [_lane_bcast(r16, i) for i in range(16)]
                t0 = jj * 16

                @plsc.parallel_loop(0, hv, 1, unroll=8)
                def k_step(k):
                    col = k * 16 + lane
                    for i in range(16):
                        plsc.addupdate_scatter(
                            acc, [rvecs[i], col],
                            buf[t0 + i, pl.ds(k * 16, 16)] * valvs[i])

                return 0

            lax.fori_loop(0, CHUNK // 16, block_step, 0)

        for j in range(NBUF):
            issue(p_lo * NBUF + j, bufs[j], sems[j])

        def quad_step(p, _):
            ch0 = p * NBUF
            for j in range(NBUF):
                wait(bufs[j], sems[j])
                compute(ch0 + j, bufs[j])
                issue(ch0 + j + NBUF, bufs[j], sems[j])
            return 0

        lax.fori_loop(p_lo, p_hi, quad_step, 0)
        for j in range(NBUF):
            wait(bufs[j], sems[j])

        pltpu.sync_copy(
            acc, out_hbm.at[pl.ds(bw * G + gr0, gpw), pl.ds(col0, HP)])


def kernel(feats, indices, values):
    B, S, H = feats.shape
    feats_r = feats.reshape(B * S, H)
    gids = indices[1].astype(jnp.int32)
    vals = values.astype(jnp.float32)

    mesh = plsc.VectorSubcoreMesh(core_axis_name="c", subcore_axis_name="s")
    run = pl.kernel(
        functools.partial(_sc_body, B, S, H),
        out_type=jax.ShapeDtypeStruct((B * G, H), jnp.float32),
        mesh=mesh,
        compiler_params=pltpu.CompilerParams(
            use_tc_tiling_on_sc=False, needs_layout_passes=False),
        scratch_types=[
            [pltpu.VMEM((CHUNK, H // NHP), jnp.float32) for _ in range(NBUF)],
            pltpu.VMEM((S,), jnp.int32),
            pltpu.VMEM((S,), jnp.float32),
            pltpu.VMEM((G // (NW // B), H // NHP), jnp.float32),
            [pltpu.SemaphoreType.DMA for _ in range(NBUF)],
        ],
    )
    out = run(feats_r, gids, vals)
    return out.reshape(B, G, H)


# speedup vs baseline: 1.1699x; 1.1699x over previous
import functools

import jax
import jax.numpy as jnp
from jax import lax
from jax.experimental import pallas as pl
from jax.experimental.pallas import tpu as pltpu
from jax.experimental.pallas import tpu_sc as plsc

G = 1024
CHUNK = 32
NBUF = 4
NHP = 2
NW = 32


def _lane_bcast(v16, i):
    return lax.gather(
        v16,
        jnp.full((16, 1), i, jnp.int32),
        lax.GatherDimensionNumbers(
            offset_dims=(),
            collapsed_slice_dims=(0,),
            start_index_map=(0,),
        ),
        slice_sizes=(1,),
        mode=lax.GatherScatterMode.PROMISE_IN_BOUNDS,
    )


def _sc_body(B, S, H, feats_hbm, gids_hbm, vals_hbm, out_hbm,
             bufs, gscan, vscan, acc, sems):
    c = lax.axis_index("c")
    s = lax.axis_index("s")
    w = c * 16 + s

    wpb = NW // B
    gpw = G // wpb
    HP = H // NHP
    hv = HP // 16
    nch = S // CHUNK

    bw = w // wpb
    gr0 = (w % wpb) * gpw
    tok_base = bw * S

    pltpu.sync_copy(gids_hbm.at[pl.ds(tok_base, S)], gscan)
    pltpu.sync_copy(vals_hbm.at[pl.ds(tok_base, S)], vscan)
    lo_t = jnp.full((16,), gr0, jnp.int32)
    hi_t = jnp.full((16,), gr0 + gpw, jnp.int32)
    one = jnp.ones((16,), jnp.int32)
    zero = jnp.zeros((16,), jnp.int32)

    def count_step(i, carry):
        lo, hi = carry
        for u in range(4):
            v = gscan[pl.ds(i * 64 + u * 16, 16)]
            lo = lo + jnp.where(v < lo_t, one, zero)
            hi = hi + jnp.where(v < hi_t, one, zero)
        return lo, hi

    lo_cnt, hi_cnt = lax.fori_loop(0, S // 64, count_step, (zero, zero))
    s_lo = jnp.int32(0)
    s_hi = jnp.int32(0)
    for j in range(16):
        s_lo = s_lo + lo_cnt[j]
        s_hi = s_hi + hi_cnt[j]

    p_lo = s_lo // (NBUF * CHUNK)
    p_hi = (s_hi + NBUF * CHUNK - 1) // (NBUF * CHUNK)
    last_ch = nch - 1
    zvec = jnp.zeros((16,), jnp.float32)
    lo16 = jnp.full((16,), s_lo, jnp.int32)
    hi16 = jnp.full((16,), s_hi, jnp.int32)
    gr0_16 = jnp.full((16,), gr0, jnp.int32)

    for hp in range(NHP):
        col0 = hp * HP

        @plsc.parallel_loop(0, gpw, 1, unroll=4)
        def zero_row(t):
            for k in range(hv):
                acc[t, pl.ds(k * 16, 16)] = zvec

        def issue(ch, buf, sem):
            chc = jnp.minimum(ch, last_ch)
            pltpu.async_copy(
                feats_hbm.at[pl.ds(tok_base + chc * CHUNK, CHUNK),
                             pl.ds(col0, HP)],
                buf, sem)

        def wait(buf, sem):
            pltpu.make_async_copy(
                feats_hbm.at[pl.ds(0, CHUNK), pl.ds(col0, HP)],
                buf, sem).wait()

        lane = lax.iota(jnp.int32, 16)

        def compute(ch, buf):
            def block_step(jj, _):
                off = ch * CHUNK + jj * 16
                g16 = gscan[pl.ds(off, 16)]
                v16 = vscan[pl.ds(off, 16)]
                tok16 = off + lane
                m = (tok16 >= lo16) & (tok16 < hi16)
                val16 = jnp.where(m, v16, 0.0)
                r16 = jnp.clip(g16 - gr0_16, 0, gpw - 1)
                valvs = [_lane_bcast(val16, i) for i in range(16)]
                rvecs = [_lane_bcast(r16, i) for i in range(16)]
                t0 = jj * 16

                @plsc.parallel_loop(0, hv, 1, unroll=4)
                def k_step(k):
                    col = k * 16 + lane
                    for i in range(16):
                        plsc.addupdate_scatter(
                            acc, [rvecs[i], col],
                            buf[t0 + i, pl.ds(k * 16, 16)] * valvs[i])

                return 0

            lax.fori_loop(0, CHUNK // 16, block_step, 0)

        for j in range(NBUF):
            issue(p_lo * NBUF + j, bufs[j], sems[j])

        def quad_step(p, _):
            ch0 = p * NBUF
            for j in range(NBUF):
                wait(bufs[j], sems[j])
                compute(ch0 + j, bufs[j])
                issue(ch0 + j + NBUF, bufs[j], sems[j])
            return 0

        lax.fori_loop(p_lo, p_hi, quad_step, 0)
        for j in range(NBUF):
            wait(bufs[j], sems[j])

        pltpu.sync_copy(
            acc, out_hbm.at[pl.ds(bw * G + gr0, gpw), pl.ds(col0, HP)])


def kernel(feats, indices, values):
    B, S, H = feats.shape
    feats_r = feats.reshape(B * S, H)
    gids = indices[1].astype(jnp.int32)
    vals = values.astype(jnp.float32)

    mesh = plsc.VectorSubcoreMesh(core_axis_name="c", subcore_axis_name="s")
    run = pl.kernel(
        functools.partial(_sc_body, B, S, H),
        out_type=jax.ShapeDtypeStruct((B * G, H), jnp.float32),
        mesh=mesh,
        compiler_params=pltpu.CompilerParams(
            use_tc_tiling_on_sc=False, needs_layout_passes=False),
        scratch_types=[
            [pltpu.VMEM((CHUNK, H // NHP), jnp.float32) for _ in range(NBUF)],
            pltpu.VMEM((S,), jnp.int32),
            pltpu.VMEM((S,), jnp.float32),
            pltpu.VMEM((G // (NW // B), H // NHP), jnp.float32),
            [pltpu.SemaphoreType.DMA for _ in range(NBUF)],
        ],
    )
    out = run(feats_r, gids, vals)
    return out.reshape(B, G, H)


# speedup vs baseline: 1.1744x; 1.0039x over previous
import functools

import jax
import jax.numpy as jnp
from jax import lax
from jax.experimental import pallas as pl
from jax.experimental.pallas import tpu as pltpu
from jax.experimental.pallas import tpu_sc as plsc

G = 1024
CHUNK = 32
NBUF = 2
NR = 2


def _lane_bcast(v16, i):
    return lax.gather(
        v16,
        jnp.full((16, 1), i, jnp.int32),
        lax.GatherDimensionNumbers(
            offset_dims=(),
            collapsed_slice_dims=(0,),
            start_index_map=(0,),
        ),
        slice_sizes=(1,),
        mode=lax.GatherScatterMode.PROMISE_IN_BOUNDS,
    )


def _sc_body(B, S, H, feats_hbm, gids_hbm, vals_hbm, out_hbm,
             bufs, gscan, vscan, acc, sems):
    c = lax.axis_index("c")
    s = lax.axis_index("s")

    gpw = G // 16
    hv = H // 16
    nch = S // CHUNK
    last_ch = nch - 1

    gr0 = s * gpw
    lo_t = jnp.full((16,), gr0, jnp.int32)
    hi_t = jnp.full((16,), gr0 + gpw, jnp.int32)
    one = jnp.ones((16,), jnp.int32)
    zero = jnp.zeros((16,), jnp.int32)
    zvec = jnp.zeros((16,), jnp.float32)
    gr0_16 = jnp.full((16,), gr0, jnp.int32)
    lane = lax.iota(jnp.int32, 16)

    for r in range(NR):
        bw = 2 * r + c
        tok_base = bw * S

        pltpu.sync_copy(gids_hbm.at[pl.ds(tok_base, S)], gscan)
        pltpu.sync_copy(vals_hbm.at[pl.ds(tok_base, S)], vscan)

        def count_step(i, carry):
            lo, hi = carry
            for u in range(4):
                v = gscan[pl.ds(i * 64 + u * 16, 16)]
                lo = lo + jnp.where(v < lo_t, one, zero)
                hi = hi + jnp.where(v < hi_t, one, zero)
            return lo, hi

        lo_cnt, hi_cnt = lax.fori_loop(0, S // 64, count_step, (zero, zero))
        s_lo = jnp.int32(0)
        s_hi = jnp.int32(0)
        for j in range(16):
            s_lo = s_lo + lo_cnt[j]
            s_hi = s_hi + hi_cnt[j]

        p_lo = s_lo // (NBUF * CHUNK)
        p_hi = (s_hi + NBUF * CHUNK - 1) // (NBUF * CHUNK)
        lo16 = jnp.full((16,), s_lo, jnp.int32)
        hi16 = jnp.full((16,), s_hi, jnp.int32)

        @plsc.parallel_loop(0, gpw, 1, unroll=4)
        def zero_row(t):
            for k in range(hv):
                acc[t, pl.ds(k * 16, 16)] = zvec

        def issue(ch, buf, sem):
            chc = jnp.minimum(ch, last_ch)
            pltpu.async_copy(
                feats_hbm.at[pl.ds(tok_base + chc * CHUNK, CHUNK)], buf, sem)

        def wait(buf, sem):
            pltpu.make_async_copy(
                feats_hbm.at[pl.ds(0, CHUNK)], buf, sem).wait()

        def compute(ch, buf):
            def block_step(jj, _):
                off = ch * CHUNK + jj * 16
                g16 = gscan[pl.ds(off, 16)]
                v16 = vscan[pl.ds(off, 16)]
                tok16 = off + lane
                m = (tok16 >= lo16) & (tok16 < hi16)
                val16 = jnp.where(m, v16, 0.0)
                r16 = jnp.clip(g16 - gr0_16, 0, gpw - 1)
                valvs = [_lane_bcast(val16, i) for i in range(16)]
                rvecs = [_lane_bcast(r16, i) for i in range(16)]
                t0 = jj * 16

                @plsc.parallel_loop(0, hv, 1, unroll=4)
                def k_step(k):
                    col = k * 16 + lane
                    for i in range(16):
                        plsc.addupdate_scatter(
                            acc, [rvecs[i], col],
                            buf[t0 + i, pl.ds(k * 16, 16)] * valvs[i])

                return 0

            lax.fori_loop(0, CHUNK // 16, block_step, 0)

        for j in range(NBUF):
            issue(p_lo * NBUF + j, bufs[j], sems[j])

        def ring_step(p, _):
            ch0 = p * NBUF
            for j in range(NBUF):
                wait(bufs[j], sems[j])
                compute(ch0 + j, bufs[j])
                issue(ch0 + j + NBUF, bufs[j], sems[j])
            return 0

        lax.fori_loop(p_lo, p_hi, ring_step, 0)
        for j in range(NBUF):
            wait(bufs[j], sems[j])

        pltpu.sync_copy(acc, out_hbm.at[pl.ds(bw * G + gr0, gpw)])


def kernel(feats, indices, values):
    B, S, H = feats.shape
    feats_r = feats.reshape(B * S, H)
    gids = indices[1].astype(jnp.int32)
    vals = values.astype(jnp.float32)

    mesh = plsc.VectorSubcoreMesh(core_axis_name="c", subcore_axis_name="s")
    run = pl.kernel(
        functools.partial(_sc_body, B, S, H),
        out_type=jax.ShapeDtypeStruct((B * G, H), jnp.float32),
        mesh=mesh,
        compiler_params=pltpu.CompilerParams(
            use_tc_tiling_on_sc=False, needs_layout_passes=False),
        scratch_types=[
            [pltpu.VMEM((CHUNK, H), jnp.float32) for _ in range(NBUF)],
            pltpu.VMEM((S,), jnp.int32),
            pltpu.VMEM((S,), jnp.float32),
            pltpu.VMEM((G // 16, H), jnp.float32),
            [pltpu.SemaphoreType.DMA for _ in range(NBUF)],
        ],
    )
    out = run(feats_r, gids, vals)
    return out.reshape(B, G, H)
